# baseline (speedup 1.0000x reference)
import jax
import jax.numpy as jnp
from jax.experimental import pallas as pl
from jax.experimental.pallas import tpu as pltpu

_ROWS = 1024
_LBL = 512
_SPLITS = (0, 512, 1024)


def _loss_body(p_hbm, s_hbm, a_hbm, out_ref, p_v, a_v, s_v, cand_v, acc_v, sems):
    p_copy = pltpu.make_async_copy(p_hbm, p_v, sems.at[0])
    a_copy = pltpu.make_async_copy(a_hbm, a_v, sems.at[1])
    s_copies = [
        pltpu.make_async_copy(
            s_hbm.at[:, pl.ds(lo, hi - lo)],
            s_v.at[:, pl.ds(lo, hi - lo)],
            sems.at[2 + k],
        )
        for k, (lo, hi) in enumerate(zip(_SPLITS[:-1], _SPLITS[1:]))
    ]
    p_copy.start()
    a_copy.start()
    for c in s_copies:
        c.start()

    p_copy.wait()
    a_copy.wait()
    inv_l = jnp.float32(1.0 / _LBL)
    cand_v[...] = (
        jnp.dot(p_v[...], a_v[...], preferred_element_type=jnp.float32) * inv_l
    )

    rs = None
    for k, (lo, hi) in enumerate(zip(_SPLITS[:-1], _SPLITS[1:])):
        s_copies[k].wait()
        s_blk = s_v[:, pl.ds(lo, hi - lo)]
        part = jnp.dot(
            s_blk,
            cand_v[pl.ds(lo, hi - lo), :],
            preferred_element_type=jnp.float32,
            precision=jax.lax.Precision.DEFAULT,
        )
        rs_part = jnp.sum(s_blk, axis=1)
        if k == 0:
            acc_v[...] = part
            rs = rs_part
        else:
            acc_v[...] += part
            rs = rs + rs_part

    diff = p_v[...] - acc_v[...]
    sq = jnp.sum(diff * diff, axis=1)
    norms = jnp.sqrt(sq)
    mask = rs != 0
    cnt = jnp.sum(mask.astype(jnp.float32))
    total = jnp.sum(jnp.where(mask, norms, jnp.float32(0.0)))
    out_ref[...] = jnp.reshape(total / cnt, (1, 1))


def kernel(predicts, similarities, adjList):
    out = pl.pallas_call(
        _loss_body,
        in_specs=[
            pl.BlockSpec(memory_space=pltpu.MemorySpace.HBM),
            pl.BlockSpec(memory_space=pltpu.MemorySpace.HBM),
            pl.BlockSpec(memory_space=pltpu.MemorySpace.HBM),
        ],
        out_specs=pl.BlockSpec(memory_space=pltpu.VMEM),
        out_shape=jax.ShapeDtypeStruct((1, 1), jnp.float32),
        scratch_shapes=[
            pltpu.VMEM((_ROWS, _LBL), jnp.float32),
            pltpu.VMEM((_LBL, _LBL), jnp.float32),
            pltpu.VMEM((_ROWS, _ROWS), jnp.float32),
            pltpu.VMEM((_ROWS, _LBL), jnp.float32),
            pltpu.VMEM((_ROWS, _LBL), jnp.float32),
            pltpu.SemaphoreType.DMA((2 + len(_SPLITS) - 1,)),
        ],
    )(predicts, similarities, adjList)
    return out[0, 0]
